# Initial kernel scaffold; baseline (speedup 1.0000x reference)
import jax
import jax.numpy as jnp
from jax.experimental import pallas as pl


def kernel(x, w_ego_root, w_ego_u, w_layer_v, w_layer_u, w_threshold, bias, edge_index, batch_nodes):
    raise NotImplementedError("write your pallas kernel here")


# speedup vs baseline: 221.4041x; 221.4041x over previous
import functools

import jax
import jax.numpy as jnp
from jax import lax
from jax.experimental import pallas as pl
from jax.experimental.pallas import tpu as pltpu
from jax.experimental.pallas import tpu_sc as plsc

N = 10000
E = 320000
D = 128
G = 2
B = 512
KB = 200
NP_ = 10240
SLOTS = 640
DUMP = 512
NTILES = 32
EC = E // NTILES
CH = 2000
CAP = EC + 16
L = 16

_sc_mesh = plsc.VectorSubcoreMesh(core_axis_name="c", subcore_axis_name="s",
                                  num_cores=2, num_subcores=16)


def _build_kernel_a(interpret=False):
    @functools.partial(
        pl.kernel,
        out_type=[
            jax.ShapeDtypeStruct((NTILES, NP_), jnp.float32),
            jax.ShapeDtypeStruct((B,), jnp.int32),
            jax.ShapeDtypeStruct((B, D), jnp.float32),
        ],
        mesh=_sc_mesh,
        compiler_params=pltpu.CompilerParams(needs_layout_passes=False, use_tc_tiling_on_sc=False),
        scratch_types=[
            pltpu.VMEM((NP_,), jnp.float32),
            pltpu.VMEM((EC,), jnp.int32),
            pltpu.VMEM((N,), jnp.int32),
            pltpu.VMEM((B,), jnp.int32),
            pltpu.VMEM((B,), jnp.int32),
            pltpu.VMEM((L,), jnp.int32),
            pltpu.VMEM((L, D), jnp.float32),
            pltpu.SemaphoreType.DMA,
        ],
        interpret=interpret,
    )
    def kern(dst_hbm, batch_hbm, x_hbm, deg_out, soe_out, xb_out,
             deg_v, dst_v, n2s_v, batch_v, soe_v, idx_v, rows_v, sem):
        cid = lax.axis_index("c")
        sid = lax.axis_index("s")
        wid = cid * 16 + sid
        zf = jnp.zeros((L,), jnp.float32)

        def zbody(i, c):
            deg_v[pl.ds(i * L, L)] = zf
            return c
        lax.fori_loop(0, NP_ // L, zbody, 0)

        pltpu.sync_copy(dst_hbm.at[pl.ds(wid * EC, EC)], dst_v)
        onesf = jnp.ones((L,), jnp.float32)

        def sbody(i, c):
            idx = dst_v[pl.ds(i * L, L)]
            plsc.addupdate_scatter(deg_v, [idx], onesf)
            return c
        lax.fori_loop(0, EC // L, sbody, 0)
        pltpu.sync_copy(deg_v, deg_out.at[wid])

        pltpu.sync_copy(batch_hbm.at[pl.ds(wid * L, L)], idx_v)
        pltpu.async_copy(x_hbm.at[idx_v], rows_v, sem).wait()
        pltpu.sync_copy(rows_v, xb_out.at[pl.ds(wid * L, L)])

        @pl.when(wid == 0)
        def _():
            pltpu.sync_copy(batch_hbm, batch_v)
            neg1 = jnp.full((L,), -1, jnp.int32)

            def mbody(i, c):
                n2s_v[pl.ds(i * L, L)] = neg1
                return c
            lax.fori_loop(0, N // L, mbody, 0)
            iota = lax.iota(jnp.int32, L)

            def scb(i, c):
                bidx = batch_v[pl.ds(i * L, L)]
                plsc.store_scatter(n2s_v, [bidx], iota + i * L)
                return c
            lax.fori_loop(0, B // L, scb, 0)

            def gab(i, c):
                bidx = batch_v[pl.ds(i * L, L)]
                soe_v[pl.ds(i * L, L)] = plsc.load_gather(n2s_v, [bidx])
                return c
            lax.fori_loop(0, B // L, gab, 0)
            pltpu.sync_copy(soe_v, soe_out)

    return kern


def _build_kernel_b(interpret=False):
    def body(x_ref, w8_ref, deg_ref, lv_ref, nimp_ref):
        xb = x_ref[...]
        w8 = w8_ref[...]
        lv_ref[...] = lax.dot_general(
            w8, xb, (((1,), (1,)), ((), ())),
            preferred_element_type=jnp.float32)
        deg = jnp.sum(deg_ref[...], axis=0, keepdims=True)
        nimp_ref[...] = 1.0 / jnp.maximum(deg, 1.0)

    return pl.pallas_call(
        body,
        grid=(10,),
        in_specs=[
            pl.BlockSpec((NP_ // 10, D), lambda i: (i, 0)),
            pl.BlockSpec((8, D), lambda i: (0, 0)),
            pl.BlockSpec((NTILES, NP_ // 10), lambda i: (0, i)),
        ],
        out_specs=[
            pl.BlockSpec((8, NP_ // 10), lambda i: (0, i)),
            pl.BlockSpec((1, NP_ // 10), lambda i: (0, i)),
        ],
        out_shape=[
            jax.ShapeDtypeStruct((8, NP_), jnp.float32),
            jax.ShapeDtypeStruct((1, NP_), jnp.float32),
        ],
        interpret=interpret,
    )


def _build_kernel_c(interpret=False):
    @functools.partial(
        pl.kernel,
        out_type=[
            jax.ShapeDtypeStruct((2, G, SLOTS, D), jnp.float32),
            jax.ShapeDtypeStruct((NTILES, G, SLOTS), jnp.float32),
        ],
        mesh=_sc_mesh,
        compiler_params=pltpu.CompilerParams(needs_layout_passes=False, use_tc_tiling_on_sc=False),
        scratch_types=[
            pltpu.VMEM((N,), jnp.float32),
            pltpu.VMEM((N,), jnp.float32),
            pltpu.VMEM((N,), jnp.float32),
            pltpu.VMEM((N,), jnp.float32),
            pltpu.VMEM((N,), jnp.float32),
            pltpu.VMEM((N,), jnp.int32),
            pltpu.VMEM((B,), jnp.int32),
            pltpu.VMEM((CH,), jnp.int32),
            pltpu.VMEM((CH,), jnp.int32),
            pltpu.VMEM((CAP,), jnp.int32),
            pltpu.VMEM((CAP,), jnp.int32),
            pltpu.VMEM((CAP,), jnp.float32),
            pltpu.VMEM((CAP,), jnp.float32),
            pltpu.VMEM((L, D), jnp.float32),
            pltpu.VMEM((L, D), jnp.float32),
            pltpu.VMEM((L, D), jnp.float32),
            pltpu.VMEM((SLOTS,), jnp.float32),
            pltpu.VMEM((SLOTS,), jnp.float32),
            pltpu.SemaphoreType.DMA,
            pltpu.VMEM_SHARED((SLOTS, D), jnp.float32),
            pltpu.VMEM_SHARED((SLOTS, D), jnp.float32),
        ],
        interpret=interpret,
    )
    def kern(src_hbm, dst_hbm, lv_hbm, nimp_hbm, batch_hbm, x_hbm, zrow_hbm,
             agg_out, pn_out,
             lu0_v, lv0_v, lu1_v, lv1_v, nimp_v, n2s_v, batch_v,
             srcc_v, dstc_v, srcR, slotR, e0R, e1R,
             rows_v, s0_v, s1_v, pn0_v, pn1_v, sem, agg0_sp, agg1_sp):
        cid = lax.axis_index("c")
        sid = lax.axis_index("s")
        wid = cid * 16 + sid
        zf = jnp.zeros((L,), jnp.float32)

        pltpu.sync_copy(lv_hbm.at[0, pl.ds(0, N)], lu0_v)
        pltpu.sync_copy(lv_hbm.at[1, pl.ds(0, N)], lv0_v)
        pltpu.sync_copy(lv_hbm.at[2, pl.ds(0, N)], lu1_v)
        pltpu.sync_copy(lv_hbm.at[3, pl.ds(0, N)], lv1_v)
        pltpu.sync_copy(nimp_hbm.at[0, pl.ds(0, N)], nimp_v)
        pltpu.sync_copy(batch_hbm, batch_v)

        neg1 = jnp.full((L,), -1, jnp.int32)

        def mbody(i, c):
            n2s_v[pl.ds(i * L, L)] = neg1
            return c
        lax.fori_loop(0, N // L, mbody, 0)
        iota = lax.iota(jnp.int32, L)

        def scb(i, c):
            bidx = batch_v[pl.ds(i * L, L)]
            plsc.store_scatter(n2s_v, [bidx], iota + i * L)
            return c
        lax.fori_loop(0, B // L, scb, 0)

        def pzb(i, c):
            pn0_v[pl.ds(i * L, L)] = zf
            pn1_v[pl.ds(i * L, L)] = zf
            return c
        lax.fori_loop(0, SLOTS // L, pzb, 0)
        rs = SLOTS // 16
        pltpu.sync_copy(zrow_hbm, agg0_sp.at[pl.ds(sid * rs, rs)])
        pltpu.sync_copy(zrow_hbm, agg1_sp.at[pl.ds(sid * rs, rs)])
        plsc.subcore_barrier()

        def chunk_body(cix, off):
            base = wid * EC + cix * CH
            pltpu.sync_copy(src_hbm.at[pl.ds(base, CH)], srcc_v)
            pltpu.sync_copy(dst_hbm.at[pl.ds(base, CH)], dstc_v)

            def gbody(i, off):
                dst16 = dstc_v[pl.ds(i * L, L)]
                slot16 = plsc.load_gather(n2s_v, [dst16])
                mask = slot16 >= 0
                cnt = jnp.sum(jnp.where(mask, 1, 0))

                @pl.when(cnt > 0)
                def _():
                    src16 = srcc_v[pl.ds(i * L, L)]
                    ni = plsc.load_gather(nimp_v, [src16])
                    a0 = (plsc.load_gather(lu0_v, [src16])
                          + plsc.load_gather(lv0_v, [dst16]))
                    a1 = (plsc.load_gather(lu1_v, [src16])
                          + plsc.load_gather(lv1_v, [dst16]))
                    e0 = (1.0 / (1.0 + jnp.exp(-a0))) * ni
                    e1 = (1.0 / (1.0 + jnp.exp(-a1))) * ni
                    slot_s = jnp.where(mask, slot16, DUMP)
                    plsc.addupdate_scatter(pn0_v, [slot_s], e0, mask=mask)
                    plsc.addupdate_scatter(pn1_v, [slot_s], e1, mask=mask)
                    plsc.store_compressed(srcR.at[pl.ds(off, L)], src16,
                                          mask=mask)
                    plsc.store_compressed(slotR.at[pl.ds(off, L)], slot_s,
                                          mask=mask)
                    plsc.store_compressed(e0R.at[pl.ds(off, L)], e0, mask=mask)
                    plsc.store_compressed(e1R.at[pl.ds(off, L)], e1, mask=mask)
                return off + cnt
            return lax.fori_loop(0, CH // L, gbody, off)

        off = lax.fori_loop(0, EC // CH, chunk_body, jnp.int32(0))

        srcR[pl.ds(off, L)] = jnp.zeros((L,), jnp.int32)
        slotR[pl.ds(off, L)] = jnp.full((L,), DUMP, jnp.int32)
        e0R[pl.ds(off, L)] = zf
        e1R[pl.ds(off, L)] = zf

        ng = (off + L - 1) // L

        def rbody(j, c):
            o = j * L
            src16 = srcR[pl.ds(o, L)]
            pltpu.async_copy(x_hbm.at[src16], rows_v, sem).wait()
            slot16 = slotR[pl.ds(o, L)]
            for k in range(L):
                kk = jnp.full((L,), o + k, jnp.int32)
                e0b = plsc.load_gather(e0R, [kk])
                e1b = plsc.load_gather(e1R, [kk])
                for dd in range(D // L):
                    v = rows_v[k, pl.ds(dd * L, L)]
                    s0_v[k, pl.ds(dd * L, L)] = v * e0b
                    s1_v[k, pl.ds(dd * L, L)] = v * e1b
            pltpu.sync_copy(s0_v, agg0_sp.at[slot16], add=True)
            pltpu.sync_copy(s1_v, agg1_sp.at[slot16], add=True)
            return c
        lax.fori_loop(0, ng, rbody, 0)

        plsc.subcore_barrier()
        pltpu.sync_copy(agg0_sp.at[pl.ds(sid * rs, rs)],
                        agg_out.at[cid, 0, pl.ds(sid * rs, rs)])
        pltpu.sync_copy(agg1_sp.at[pl.ds(sid * rs, rs)],
                        agg_out.at[cid, 1, pl.ds(sid * rs, rs)])
        pltpu.sync_copy(pn0_v, pn_out.at[wid, 0])
        pltpu.sync_copy(pn1_v, pn_out.at[wid, 1])

    return kern


def _build_kernel_d(interpret=False):
    def body(xb_ref, soe_col_ref, aggp_ref, pnp_ref, wr_ref,
             wu_ref, wt_ref, bias_ref, pall_ref, p_ref):
        f32 = jnp.float32
        xb = xb_ref[...]
        soe_col = soe_col_ref[...]
        aggp = aggp_ref[...]
        pn = jnp.sum(pnp_ref[...], axis=0)
        wr = wr_ref[...]
        wu = wu_ref[...]
        wt = wt_ref[...]
        bias = bias_ref[...]

        slot_iota = lax.broadcasted_iota(jnp.int32, (B, SLOTS), 1)
        onehot = (soe_col == slot_iota).astype(f32)
        wt8 = jnp.concatenate([wt, jnp.zeros((8 - G, D), f32)], axis=0)
        thr_all = lax.dot_general(xb, wt8, (((1,), (1,)), ((), ())),
                                  preferred_element_type=f32)

        p_cols = []
        pall_cols = []
        for g in range(G):
            agg_g = aggp[0, g] + aggp[1, g]
            agg_e = lax.dot_general(onehot, agg_g,
                                    (((1,), (0,)), ((), ())),
                                    preferred_element_type=f32,
                                    precision=lax.Precision.HIGHEST)
            a = agg_e * wu[g:g + 1]
            h = xb * wr[g:g + 1] + bias[g:g + 1]
            num = jnp.sum(a * h, axis=1, keepdims=True)
            na = jnp.sqrt(jnp.sum(a * a, axis=1, keepdims=True))
            nh = jnp.sqrt(jnp.sum(h * h, axis=1, keepdims=True))
            cos = num / (na * nh + 1e-6)
            pn_e = lax.dot_general(onehot, pn[g:g + 1],
                                   (((1,), (1,)), ((), ())),
                                   preferred_element_type=f32,
                                   precision=lax.Precision.HIGHEST)
            pcol = 0.5 * cos + 0.5 * jax.nn.sigmoid(pn_e)
            thr = jax.nn.sigmoid(thr_all[:, g:g + 1])
            pall_g = jnp.where(pcol > thr, pcol, 0.0)
            pall_cols.append(pall_g)

        pall_ref[...] = jnp.concatenate(pall_cols, axis=1)
        p_ref[...] = pall_cols[0] + pall_cols[1]

    return pl.pallas_call(
        body,
        out_shape=[
            jax.ShapeDtypeStruct((B, G), jnp.float32),
            jax.ShapeDtypeStruct((B, 1), jnp.float32),
        ],
        interpret=interpret,
    )


def _build_kernel_e(interpret=False):
    def body(pc_ref, pr_ref, sc_ref, sr_ref, bp_ref, bi_ref):
        f32 = jnp.float32
        p_col = pc_ref[...]
        p_row = pr_ref[...]
        soe_col = sc_ref[...]
        soe_row = sr_ref[...]
        ii = lax.broadcasted_iota(jnp.int32, (B, B), 0)
        jj = lax.broadcasted_iota(jnp.int32, (B, B), 1)
        same_slot = soe_col == soe_row
        idx_lt = ii < jj
        beats = (jnp.logical_not(same_slot)
                 & ((p_col > p_row) | ((p_col == p_row) & idx_lt))
                 | (same_slot & idx_lt))
        rank_row = jnp.sum(beats.astype(f32), axis=0, keepdims=True)
        r_iota = lax.broadcasted_iota(jnp.int32, (KB, B), 0).astype(f32)
        sel = (r_iota == rank_row).astype(f32)
        bp_ref[...] = jnp.sum(sel * p_row, axis=1, keepdims=True)
        idx_row = lax.broadcasted_iota(jnp.int32, (1, B), 1).astype(f32)
        bi_ref[...] = jnp.sum(sel * idx_row, axis=1,
                              keepdims=True).astype(jnp.int32)

    return pl.pallas_call(
        body,
        out_shape=[
            jax.ShapeDtypeStruct((KB, 1), jnp.float32),
            jax.ShapeDtypeStruct((KB, 1), jnp.int32),
        ],
        interpret=interpret,
    )


_kernel_a = _build_kernel_a()
_kernel_b = _build_kernel_b()
_kernel_c = _build_kernel_c()
_kernel_d = _build_kernel_d()
_kernel_e = _build_kernel_e()


def kernel(x, w_ego_root, w_ego_u, w_layer_v, w_layer_u, w_threshold, bias,
           edge_index, batch_nodes):
    src = edge_index[0]
    dst = edge_index[1]
    deg_parts, soe, xb = _kernel_a(dst, batch_nodes, x)
    w8 = jnp.concatenate(
        [w_layer_u[0:1], w_layer_v[0:1], w_layer_u[1:2], w_layer_v[1:2],
         jnp.zeros((4, D), jnp.float32)], axis=0)
    lv8, nimp = _kernel_b(x, w8, deg_parts)
    zrows = jnp.zeros((SLOTS // 16, D), jnp.float32)
    agg_parts, pn_parts = _kernel_c(src, dst, lv8, nimp, batch_nodes, x,
                                    zrows)
    pall_col, p_col = _kernel_d(xb, soe.reshape(B, 1), agg_parts, pn_parts,
                                w_ego_root, w_ego_u, w_threshold, bias)
    bp, bi = _kernel_e(p_col, p_col.reshape(1, B),
                       soe.reshape(B, 1), soe.reshape(1, B))
    return (jnp.transpose(pall_col), bp.reshape(KB), bi.reshape(KB))


# speedup vs baseline: 242.2728x; 1.0943x over previous
import functools

import jax
import jax.numpy as jnp
from jax import lax
from jax.experimental import pallas as pl
from jax.experimental.pallas import tpu as pltpu
from jax.experimental.pallas import tpu_sc as plsc

N = 10000
E = 320000
D = 128
G = 2
B = 512
KB = 200
NP_ = 10240
SLOTS = 640
DUMP = 512
NTILES = 32
EC = E // NTILES
CH = 2000
CAP = EC + 16
L = 16

_sc_mesh = plsc.VectorSubcoreMesh(core_axis_name="c", subcore_axis_name="s",
                                  num_cores=2, num_subcores=16)


def _build_kernel_a(interpret=False):
    @functools.partial(
        pl.kernel,
        out_type=[
            jax.ShapeDtypeStruct((NTILES, NP_), jnp.float32),
            jax.ShapeDtypeStruct((B,), jnp.int32),
            jax.ShapeDtypeStruct((B, D), jnp.float32),
        ],
        mesh=_sc_mesh,
        compiler_params=pltpu.CompilerParams(needs_layout_passes=False, use_tc_tiling_on_sc=False),
        scratch_types=[
            pltpu.VMEM((NP_,), jnp.float32),
            pltpu.VMEM((EC,), jnp.int32),
            pltpu.VMEM((N,), jnp.int32),
            pltpu.VMEM((B,), jnp.int32),
            pltpu.VMEM((B,), jnp.int32),
            pltpu.VMEM((L,), jnp.int32),
            pltpu.VMEM((L, D), jnp.float32),
            pltpu.SemaphoreType.DMA,
        ],
        interpret=interpret,
    )
    def kern(dst_hbm, batch_hbm, x_hbm, deg_out, soe_out, xb_out,
             deg_v, dst_v, n2s_v, batch_v, soe_v, idx_v, rows_v, sem):
        cid = lax.axis_index("c")
        sid = lax.axis_index("s")
        wid = cid * 16 + sid
        zf = jnp.zeros((L,), jnp.float32)

        def zbody(i, c):
            deg_v[pl.ds(i * L, L)] = zf
            return c
        lax.fori_loop(0, NP_ // L, zbody, 0)

        pltpu.sync_copy(dst_hbm.at[pl.ds(wid * EC, EC)], dst_v)
        onesf = jnp.ones((L,), jnp.float32)

        def sbody(i, c):
            idx = dst_v[pl.ds(i * L, L)]
            plsc.addupdate_scatter(deg_v, [idx], onesf)
            return c
        lax.fori_loop(0, EC // L, sbody, 0)
        pltpu.sync_copy(deg_v, deg_out.at[wid])

        pltpu.sync_copy(batch_hbm.at[pl.ds(wid * L, L)], idx_v)
        pltpu.async_copy(x_hbm.at[idx_v], rows_v, sem).wait()
        pltpu.sync_copy(rows_v, xb_out.at[pl.ds(wid * L, L)])

        @pl.when(wid == 0)
        def _():
            pltpu.sync_copy(batch_hbm, batch_v)
            neg1 = jnp.full((L,), -1, jnp.int32)

            def mbody(i, c):
                n2s_v[pl.ds(i * L, L)] = neg1
                return c
            lax.fori_loop(0, N // L, mbody, 0)
            iota = lax.iota(jnp.int32, L)

            def scb(i, c):
                bidx = batch_v[pl.ds(i * L, L)]
                plsc.store_scatter(n2s_v, [bidx], iota + i * L)
                return c
            lax.fori_loop(0, B // L, scb, 0)

            def gab(i, c):
                bidx = batch_v[pl.ds(i * L, L)]
                soe_v[pl.ds(i * L, L)] = plsc.load_gather(n2s_v, [bidx])
                return c
            lax.fori_loop(0, B // L, gab, 0)
            pltpu.sync_copy(soe_v, soe_out)

    return kern


def _build_kernel_b(interpret=False):
    def body(x_ref, w8_ref, deg_ref, lv_ref, nimp_ref):
        xb = x_ref[...]
        w8 = w8_ref[...]
        lv_ref[...] = lax.dot_general(
            w8, xb, (((1,), (1,)), ((), ())),
            preferred_element_type=jnp.float32)
        deg = jnp.sum(deg_ref[...], axis=0, keepdims=True)
        nimp_ref[...] = 1.0 / jnp.maximum(deg, 1.0)

    return pl.pallas_call(
        body,
        grid=(10,),
        in_specs=[
            pl.BlockSpec((NP_ // 10, D), lambda i: (i, 0)),
            pl.BlockSpec((8, D), lambda i: (0, 0)),
            pl.BlockSpec((NTILES, NP_ // 10), lambda i: (0, i)),
        ],
        out_specs=[
            pl.BlockSpec((8, NP_ // 10), lambda i: (0, i)),
            pl.BlockSpec((1, NP_ // 10), lambda i: (0, i)),
        ],
        out_shape=[
            jax.ShapeDtypeStruct((8, NP_), jnp.float32),
            jax.ShapeDtypeStruct((1, NP_), jnp.float32),
        ],
        interpret=interpret,
    )


def _build_kernel_c(interpret=False):
    @functools.partial(
        pl.kernel,
        out_type=[
            jax.ShapeDtypeStruct((2, G, SLOTS, D), jnp.float32),
            jax.ShapeDtypeStruct((NTILES, G, SLOTS), jnp.float32),
        ],
        mesh=_sc_mesh,
        compiler_params=pltpu.CompilerParams(needs_layout_passes=False, use_tc_tiling_on_sc=False),
        scratch_types=[
            pltpu.VMEM((N,), jnp.float32),
            pltpu.VMEM((N,), jnp.float32),
            pltpu.VMEM((N,), jnp.float32),
            pltpu.VMEM((N,), jnp.float32),
            pltpu.VMEM((N,), jnp.float32),
            pltpu.VMEM((N,), jnp.int32),
            pltpu.VMEM((B,), jnp.int32),
            pltpu.VMEM((CH,), jnp.int32),
            pltpu.VMEM((CH,), jnp.int32),
            pltpu.VMEM((CAP,), jnp.int32),
            pltpu.VMEM((CAP,), jnp.int32),
            pltpu.VMEM((CAP,), jnp.float32),
            pltpu.VMEM((CAP,), jnp.float32),
            pltpu.VMEM((L, D), jnp.float32),
            pltpu.VMEM((L, D), jnp.float32),
            pltpu.VMEM((L, D), jnp.float32),
            pltpu.VMEM((L, D), jnp.float32),
            pltpu.VMEM((SLOTS,), jnp.float32),
            pltpu.VMEM((SLOTS,), jnp.float32),
            pltpu.SemaphoreType.DMA,
            pltpu.SemaphoreType.DMA,
            pltpu.SemaphoreType.DMA,
            pltpu.VMEM_SHARED((SLOTS, D), jnp.float32),
            pltpu.VMEM_SHARED((SLOTS, D), jnp.float32),
        ],
        interpret=interpret,
    )
    def kern(src_hbm, dst_hbm, lv_hbm, nimp_hbm, batch_hbm, x_hbm, zrow_hbm,
             agg_out, pn_out,
             lu0_v, lv0_v, lu1_v, lv1_v, nimp_v, n2s_v, batch_v,
             srcc_v, dstc_v, srcR, slotR, e0R, e1R,
             rows0_v, rows1_v, s0_v, s1_v, pn0_v, pn1_v,
             sem, gsem0, gsem1, agg0_sp, agg1_sp):
        cid = lax.axis_index("c")
        sid = lax.axis_index("s")
        wid = cid * 16 + sid
        zf = jnp.zeros((L,), jnp.float32)

        pltpu.sync_copy(lv_hbm.at[0, pl.ds(0, N)], lu0_v)
        pltpu.sync_copy(lv_hbm.at[1, pl.ds(0, N)], lv0_v)
        pltpu.sync_copy(lv_hbm.at[2, pl.ds(0, N)], lu1_v)
        pltpu.sync_copy(lv_hbm.at[3, pl.ds(0, N)], lv1_v)
        pltpu.sync_copy(nimp_hbm.at[0, pl.ds(0, N)], nimp_v)
        pltpu.sync_copy(batch_hbm, batch_v)

        neg1 = jnp.full((L,), -1, jnp.int32)

        def mbody(i, c):
            n2s_v[pl.ds(i * L, L)] = neg1
            return c
        lax.fori_loop(0, N // L, mbody, 0)
        iota = lax.iota(jnp.int32, L)

        def scb(i, c):
            bidx = batch_v[pl.ds(i * L, L)]
            plsc.store_scatter(n2s_v, [bidx], iota + i * L)
            return c
        lax.fori_loop(0, B // L, scb, 0)

        def pzb(i, c):
            pn0_v[pl.ds(i * L, L)] = zf
            pn1_v[pl.ds(i * L, L)] = zf
            return c
        lax.fori_loop(0, SLOTS // L, pzb, 0)
        rs = SLOTS // 16
        pltpu.sync_copy(zrow_hbm, agg0_sp.at[pl.ds(sid * rs, rs)])
        pltpu.sync_copy(zrow_hbm, agg1_sp.at[pl.ds(sid * rs, rs)])
        plsc.subcore_barrier()

        def chunk_body(cix, off):
            base = wid * EC + cix * CH
            pltpu.sync_copy(src_hbm.at[pl.ds(base, CH)], srcc_v)
            pltpu.sync_copy(dst_hbm.at[pl.ds(base, CH)], dstc_v)

            def gbody(i, off):
                dst16 = dstc_v[pl.ds(i * L, L)]
                slot16 = plsc.load_gather(n2s_v, [dst16])
                mask = slot16 >= 0
                cnt = jnp.sum(jnp.where(mask, 1, 0))

                @pl.when(cnt > 0)
                def _():
                    src16 = srcc_v[pl.ds(i * L, L)]
                    ni = plsc.load_gather(nimp_v, [src16])
                    a0 = (plsc.load_gather(lu0_v, [src16])
                          + plsc.load_gather(lv0_v, [dst16]))
                    a1 = (plsc.load_gather(lu1_v, [src16])
                          + plsc.load_gather(lv1_v, [dst16]))
                    e0 = (1.0 / (1.0 + jnp.exp(-a0))) * ni
                    e1 = (1.0 / (1.0 + jnp.exp(-a1))) * ni
                    slot_s = jnp.where(mask, slot16, DUMP)
                    plsc.addupdate_scatter(pn0_v, [slot_s], e0, mask=mask)
                    plsc.addupdate_scatter(pn1_v, [slot_s], e1, mask=mask)
                    plsc.store_compressed(srcR.at[pl.ds(off, L)], src16,
                                          mask=mask)
                    plsc.store_compressed(slotR.at[pl.ds(off, L)], slot_s,
                                          mask=mask)
                    plsc.store_compressed(e0R.at[pl.ds(off, L)], e0, mask=mask)
                    plsc.store_compressed(e1R.at[pl.ds(off, L)], e1, mask=mask)
                return off + cnt
            return lax.fori_loop(0, CH // L, gbody, off)

        off = lax.fori_loop(0, EC // CH, chunk_body, jnp.int32(0))

        srcR[pl.ds(off, L)] = jnp.zeros((L,), jnp.int32)
        slotR[pl.ds(off, L)] = jnp.full((L,), DUMP, jnp.int32)
        e0R[pl.ds(off, L)] = zf
        e1R[pl.ds(off, L)] = zf

        ng = (off + L - 1) // L

        def _start(j, buf, gsem):
            src16 = srcR[pl.ds(j * L, L)]
            pltpu.async_copy(x_hbm.at[src16], buf, gsem)

        def _drain(j, buf, gsem):
            src16 = srcR[pl.ds(j * L, L)]
            pltpu.make_async_copy(x_hbm.at[src16], buf, gsem).wait()

        def _consume(j, buf):
            o = j * L
            slot16 = slotR[pl.ds(o, L)]
            for k in range(L):
                kk = jnp.full((L,), o + k, jnp.int32)
                e0b = plsc.load_gather(e0R, [kk])
                e1b = plsc.load_gather(e1R, [kk])
                for dd in range(D // L):
                    v = buf[k, pl.ds(dd * L, L)]
                    s0_v[k, pl.ds(dd * L, L)] = v * e0b
                    s1_v[k, pl.ds(dd * L, L)] = v * e1b
            pltpu.sync_copy(s0_v, agg0_sp.at[slot16], add=True)
            pltpu.sync_copy(s1_v, agg1_sp.at[slot16], add=True)

        @pl.when(ng > 0)
        def _():
            _start(0, rows0_v, gsem0)

        def rbody(jj, c):
            j0 = jj * 2
            j1 = j0 + 1

            @pl.when(j1 < ng)
            def _():
                _start(j1, rows1_v, gsem1)

            @pl.when(j0 < ng)
            def _():
                _drain(j0, rows0_v, gsem0)
                _consume(j0, rows0_v)

            @pl.when(j0 + 2 < ng)
            def _():
                _start(j0 + 2, rows0_v, gsem0)

            @pl.when(j1 < ng)
            def _():
                _drain(j1, rows1_v, gsem1)
                _consume(j1, rows1_v)
            return c
        lax.fori_loop(0, (ng + 1) // 2, rbody, 0)

        plsc.subcore_barrier()
        pltpu.sync_copy(agg0_sp.at[pl.ds(sid * rs, rs)],
                        agg_out.at[cid, 0, pl.ds(sid * rs, rs)])
        pltpu.sync_copy(agg1_sp.at[pl.ds(sid * rs, rs)],
                        agg_out.at[cid, 1, pl.ds(sid * rs, rs)])
        pltpu.sync_copy(pn0_v, pn_out.at[wid, 0])
        pltpu.sync_copy(pn1_v, pn_out.at[wid, 1])

    return kern


def _build_kernel_d(interpret=False):
    def body(xb_ref, soe_col_ref, aggp_ref, pnp_ref, wr_ref,
             wu_ref, wt_ref, bias_ref, pall_ref, p_ref):
        f32 = jnp.float32
        xb = xb_ref[...]
        soe_col = soe_col_ref[...]
        aggp = aggp_ref[...]
        pn = jnp.sum(pnp_ref[...], axis=0)
        wr = wr_ref[...]
        wu = wu_ref[...]
        wt = wt_ref[...]
        bias = bias_ref[...]

        slot_iota = lax.broadcasted_iota(jnp.int32, (B, SLOTS), 1)
        onehot = (soe_col == slot_iota).astype(f32)
        wt8 = jnp.concatenate([wt, jnp.zeros((8 - G, D), f32)], axis=0)
        thr_all = lax.dot_general(xb, wt8, (((1,), (1,)), ((), ())),
                                  preferred_element_type=f32)

        p_cols = []
        pall_cols = []
        for g in range(G):
            agg_g = aggp[0, g] + aggp[1, g]
            agg_e = lax.dot_general(onehot, agg_g,
                                    (((1,), (0,)), ((), ())),
                                    preferred_element_type=f32,
                                    precision=lax.Precision.HIGHEST)
            a = agg_e * wu[g:g + 1]
            h = xb * wr[g:g + 1] + bias[g:g + 1]
            num = jnp.sum(a * h, axis=1, keepdims=True)
            na = jnp.sqrt(jnp.sum(a * a, axis=1, keepdims=True))
            nh = jnp.sqrt(jnp.sum(h * h, axis=1, keepdims=True))
            cos = num / (na * nh + 1e-6)
            pn_e = lax.dot_general(onehot, pn[g:g + 1],
                                   (((1,), (1,)), ((), ())),
                                   preferred_element_type=f32,
                                   precision=lax.Precision.HIGHEST)
            pcol = 0.5 * cos + 0.5 * jax.nn.sigmoid(pn_e)
            thr = jax.nn.sigmoid(thr_all[:, g:g + 1])
            pall_g = jnp.where(pcol > thr, pcol, 0.0)
            pall_cols.append(pall_g)

        pall_ref[...] = jnp.concatenate(pall_cols, axis=1)
        p_ref[...] = pall_cols[0] + pall_cols[1]

    return pl.pallas_call(
        body,
        out_shape=[
            jax.ShapeDtypeStruct((B, G), jnp.float32),
            jax.ShapeDtypeStruct((B, 1), jnp.float32),
        ],
        interpret=interpret,
    )


def _build_kernel_e(interpret=False):
    def body(pc_ref, pr_ref, sc_ref, sr_ref, bp_ref, bi_ref):
        f32 = jnp.float32
        p_col = pc_ref[...]
        p_row = pr_ref[...]
        soe_col = sc_ref[...]
        soe_row = sr_ref[...]
        ii = lax.broadcasted_iota(jnp.int32, (B, B), 0)
        jj = lax.broadcasted_iota(jnp.int32, (B, B), 1)
        same_slot = soe_col == soe_row
        idx_lt = ii < jj
        beats = (jnp.logical_not(same_slot)
                 & ((p_col > p_row) | ((p_col == p_row) & idx_lt))
                 | (same_slot & idx_lt))
        rank_row = jnp.sum(beats.astype(f32), axis=0, keepdims=True)
        r_iota = lax.broadcasted_iota(jnp.int32, (KB, B), 0).astype(f32)
        sel = (r_iota == rank_row).astype(f32)
        bp_ref[...] = jnp.sum(sel * p_row, axis=1, keepdims=True)
        idx_row = lax.broadcasted_iota(jnp.int32, (1, B), 1).astype(f32)
        bi_ref[...] = jnp.sum(sel * idx_row, axis=1,
                              keepdims=True).astype(jnp.int32)

    return pl.pallas_call(
        body,
        out_shape=[
            jax.ShapeDtypeStruct((KB, 1), jnp.float32),
            jax.ShapeDtypeStruct((KB, 1), jnp.int32),
        ],
        interpret=interpret,
    )


_kernel_a = _build_kernel_a()
_kernel_b = _build_kernel_b()
_kernel_c = _build_kernel_c()
_kernel_d = _build_kernel_d()
_kernel_e = _build_kernel_e()


def kernel(x, w_ego_root, w_ego_u, w_layer_v, w_layer_u, w_threshold, bias,
           edge_index, batch_nodes):
    src = edge_index[0]
    dst = edge_index[1]
    deg_parts, soe, xb = _kernel_a(dst, batch_nodes, x)
    w8 = jnp.concatenate(
        [w_layer_u[0:1], w_layer_v[0:1], w_layer_u[1:2], w_layer_v[1:2],
         jnp.zeros((4, D), jnp.float32)], axis=0)
    lv8, nimp = _kernel_b(x, w8, deg_parts)
    zrows = jnp.zeros((SLOTS // 16, D), jnp.float32)
    agg_parts, pn_parts = _kernel_c(src, dst, lv8, nimp, batch_nodes, x,
                                    zrows)
    pall_col, p_col = _kernel_d(xb, soe.reshape(B, 1), agg_parts, pn_parts,
                                w_ego_root, w_ego_u, w_threshold, bias)
    bp, bi = _kernel_e(p_col, p_col.reshape(1, B),
                       soe.reshape(B, 1), soe.reshape(1, B))
    return (jnp.transpose(pall_col), bp.reshape(KB), bi.reshape(KB))


# speedup vs baseline: 249.6808x; 1.0306x over previous
import functools

import jax
import jax.numpy as jnp
from jax import lax
from jax.experimental import pallas as pl
from jax.experimental.pallas import tpu as pltpu
from jax.experimental.pallas import tpu_sc as plsc

N = 10000
E = 320000
D = 128
G = 2
B = 512
KB = 200
NP_ = 10240
SLOTS = 640
DUMP = 512
NTILES = 32
EC = E // NTILES
CH = 2000
CAP = EC + 16
L = 16

_sc_mesh = plsc.VectorSubcoreMesh(core_axis_name="c", subcore_axis_name="s",
                                  num_cores=2, num_subcores=16)


def _build_kernel_a(interpret=False):
    @functools.partial(
        pl.kernel,
        out_type=[
            jax.ShapeDtypeStruct((NTILES, NP_), jnp.float32),
            jax.ShapeDtypeStruct((B,), jnp.int32),
            jax.ShapeDtypeStruct((B, D), jnp.float32),
        ],
        mesh=_sc_mesh,
        compiler_params=pltpu.CompilerParams(needs_layout_passes=False, use_tc_tiling_on_sc=False),
        scratch_types=[
            pltpu.VMEM((NP_,), jnp.float32),
            pltpu.VMEM((EC,), jnp.int32),
            pltpu.VMEM((N,), jnp.int32),
            pltpu.VMEM((B,), jnp.int32),
            pltpu.VMEM((B,), jnp.int32),
            pltpu.VMEM((L,), jnp.int32),
            pltpu.VMEM((L, D), jnp.float32),
            pltpu.SemaphoreType.DMA,
        ],
        interpret=interpret,
    )
    def kern(dst_hbm, batch_hbm, x_hbm, deg_out, soe_out, xb_out,
             deg_v, dst_v, n2s_v, batch_v, soe_v, idx_v, rows_v, sem):
        cid = lax.axis_index("c")
        sid = lax.axis_index("s")
        wid = cid * 16 + sid
        zf = jnp.zeros((L,), jnp.float32)

        def zbody(i, c):
            deg_v[pl.ds(i * L, L)] = zf
            return c
        lax.fori_loop(0, NP_ // L, zbody, 0)

        pltpu.sync_copy(dst_hbm.at[pl.ds(wid * EC, EC)], dst_v)
        onesf = jnp.ones((L,), jnp.float32)

        def sbody(i, c):
            idx = dst_v[pl.ds(i * L, L)]
            plsc.addupdate_scatter(deg_v, [idx], onesf)
            return c
        lax.fori_loop(0, EC // L, sbody, 0)
        pltpu.sync_copy(deg_v, deg_out.at[wid])

        pltpu.sync_copy(batch_hbm.at[pl.ds(wid * L, L)], idx_v)
        pltpu.async_copy(x_hbm.at[idx_v], rows_v, sem).wait()
        pltpu.sync_copy(rows_v, xb_out.at[pl.ds(wid * L, L)])

        @pl.when(wid == 0)
        def _():
            pltpu.sync_copy(batch_hbm, batch_v)
            neg1 = jnp.full((L,), -1, jnp.int32)

            def mbody(i, c):
                n2s_v[pl.ds(i * L, L)] = neg1
                return c
            lax.fori_loop(0, N // L, mbody, 0)
            iota = lax.iota(jnp.int32, L)

            def scb(i, c):
                bidx = batch_v[pl.ds(i * L, L)]
                plsc.store_scatter(n2s_v, [bidx], iota + i * L)
                return c
            lax.fori_loop(0, B // L, scb, 0)

            def gab(i, c):
                bidx = batch_v[pl.ds(i * L, L)]
                soe_v[pl.ds(i * L, L)] = plsc.load_gather(n2s_v, [bidx])
                return c
            lax.fori_loop(0, B // L, gab, 0)
            pltpu.sync_copy(soe_v, soe_out)

    return kern


def _build_kernel_b(interpret=False):
    def body(x_ref, w8_ref, deg_ref, lv_ref, nimp_ref):
        xb = x_ref[...]
        w8 = w8_ref[...]
        lv_ref[...] = lax.dot_general(
            w8, xb, (((1,), (1,)), ((), ())),
            preferred_element_type=jnp.float32)
        deg = jnp.sum(deg_ref[...], axis=0, keepdims=True)
        nimp_ref[...] = 1.0 / jnp.maximum(deg, 1.0)

    return pl.pallas_call(
        body,
        grid=(10,),
        in_specs=[
            pl.BlockSpec((NP_ // 10, D), lambda i: (i, 0)),
            pl.BlockSpec((8, D), lambda i: (0, 0)),
            pl.BlockSpec((NTILES, NP_ // 10), lambda i: (0, i)),
        ],
        out_specs=[
            pl.BlockSpec((8, NP_ // 10), lambda i: (0, i)),
            pl.BlockSpec((1, NP_ // 10), lambda i: (0, i)),
        ],
        out_shape=[
            jax.ShapeDtypeStruct((8, NP_), jnp.float32),
            jax.ShapeDtypeStruct((1, NP_), jnp.float32),
        ],
        interpret=interpret,
    )


def _build_kernel_c(interpret=False):
    @functools.partial(
        pl.kernel,
        out_type=[
            jax.ShapeDtypeStruct((2, G, SLOTS, D), jnp.float32),
            jax.ShapeDtypeStruct((NTILES, G, SLOTS), jnp.float32),
        ],
        mesh=_sc_mesh,
        compiler_params=pltpu.CompilerParams(needs_layout_passes=False, use_tc_tiling_on_sc=False),
        scratch_types=[
            pltpu.VMEM((N,), jnp.float32),
            pltpu.VMEM((N,), jnp.float32),
            pltpu.VMEM((N,), jnp.float32),
            pltpu.VMEM((N,), jnp.float32),
            pltpu.VMEM((N,), jnp.float32),
            pltpu.VMEM((N,), jnp.int32),
            pltpu.VMEM((B,), jnp.int32),
            pltpu.VMEM((CH,), jnp.int32),
            pltpu.VMEM((CH,), jnp.int32),
            pltpu.VMEM((CAP,), jnp.int32),
            pltpu.VMEM((CAP,), jnp.int32),
            pltpu.VMEM((CAP,), jnp.float32),
            pltpu.VMEM((CAP,), jnp.float32),
            pltpu.VMEM((L, D), jnp.float32),
            pltpu.VMEM((L, D), jnp.float32),
            pltpu.VMEM((L, D), jnp.float32),
            pltpu.VMEM((L, D), jnp.float32),
            pltpu.VMEM((SLOTS,), jnp.float32),
            pltpu.VMEM((SLOTS,), jnp.float32),
            pltpu.SemaphoreType.DMA,
            pltpu.SemaphoreType.DMA,
            pltpu.SemaphoreType.DMA,
            pltpu.VMEM_SHARED((SLOTS, D), jnp.float32),
            pltpu.VMEM_SHARED((SLOTS, D), jnp.float32),
        ],
        interpret=interpret,
    )
    def kern(src_hbm, dst_hbm, lv_hbm, nimp_hbm, batch_hbm, x_hbm, zrow_hbm,
             agg_out, pn_out,
             lu0_v, lv0_v, lu1_v, lv1_v, nimp_v, n2s_v, batch_v,
             srcc_v, dstc_v, srcR, slotR, e0R, e1R,
             rows0_v, rows1_v, s0_v, s1_v, pn0_v, pn1_v,
             sem, gsem0, gsem1, agg0_sp, agg1_sp):
        cid = lax.axis_index("c")
        sid = lax.axis_index("s")
        wid = cid * 16 + sid
        zf = jnp.zeros((L,), jnp.float32)

        pltpu.async_copy(lv_hbm.at[0, pl.ds(0, N)], lu0_v, sem)
        pltpu.async_copy(lv_hbm.at[1, pl.ds(0, N)], lv0_v, sem)
        pltpu.async_copy(lv_hbm.at[2, pl.ds(0, N)], lu1_v, sem)
        pltpu.async_copy(lv_hbm.at[3, pl.ds(0, N)], lv1_v, sem)
        pltpu.async_copy(nimp_hbm.at[0, pl.ds(0, N)], nimp_v, sem)
        pltpu.async_copy(batch_hbm, batch_v, sem)
        pltpu.make_async_copy(lv_hbm.at[0, pl.ds(0, N)], lu0_v, sem).wait()
        pltpu.make_async_copy(lv_hbm.at[1, pl.ds(0, N)], lv0_v, sem).wait()
        pltpu.make_async_copy(lv_hbm.at[2, pl.ds(0, N)], lu1_v, sem).wait()
        pltpu.make_async_copy(lv_hbm.at[3, pl.ds(0, N)], lv1_v, sem).wait()
        pltpu.make_async_copy(nimp_hbm.at[0, pl.ds(0, N)], nimp_v, sem).wait()
        pltpu.make_async_copy(batch_hbm, batch_v, sem).wait()

        neg1 = jnp.full((L,), -1, jnp.int32)

        def mbody(i, c):
            n2s_v[pl.ds(i * L, L)] = neg1
            return c
        lax.fori_loop(0, N // L, mbody, 0)
        iota = lax.iota(jnp.int32, L)

        def scb(i, c):
            bidx = batch_v[pl.ds(i * L, L)]
            plsc.store_scatter(n2s_v, [bidx], iota + i * L)
            return c
        lax.fori_loop(0, B // L, scb, 0)

        def pzb(i, c):
            pn0_v[pl.ds(i * L, L)] = zf
            pn1_v[pl.ds(i * L, L)] = zf
            return c
        lax.fori_loop(0, SLOTS // L, pzb, 0)
        rs = SLOTS // 16
        pltpu.sync_copy(zrow_hbm, agg0_sp.at[pl.ds(sid * rs, rs)])
        pltpu.sync_copy(zrow_hbm, agg1_sp.at[pl.ds(sid * rs, rs)])
        plsc.subcore_barrier()

        def chunk_body(cix, off):
            base = wid * EC + cix * CH
            pltpu.async_copy(src_hbm.at[pl.ds(base, CH)], srcc_v, sem)
            pltpu.async_copy(dst_hbm.at[pl.ds(base, CH)], dstc_v, sem)
            pltpu.make_async_copy(src_hbm.at[pl.ds(base, CH)], srcc_v,
                                  sem).wait()
            pltpu.make_async_copy(dst_hbm.at[pl.ds(base, CH)], dstc_v,
                                  sem).wait()

            def gbody(i, off):
                dst16 = dstc_v[pl.ds(i * L, L)]
                slot16 = plsc.load_gather(n2s_v, [dst16])
                mask = slot16 >= 0
                cnt = jnp.sum(jnp.where(mask, 1, 0))

                @pl.when(cnt > 0)
                def _():
                    src16 = srcc_v[pl.ds(i * L, L)]
                    ni = plsc.load_gather(nimp_v, [src16])
                    a0 = (plsc.load_gather(lu0_v, [src16])
                          + plsc.load_gather(lv0_v, [dst16]))
                    a1 = (plsc.load_gather(lu1_v, [src16])
                          + plsc.load_gather(lv1_v, [dst16]))
                    e0 = (1.0 / (1.0 + jnp.exp(-a0))) * ni
                    e1 = (1.0 / (1.0 + jnp.exp(-a1))) * ni
                    slot_s = jnp.where(mask, slot16, DUMP)
                    plsc.addupdate_scatter(pn0_v, [slot_s], e0, mask=mask)
                    plsc.addupdate_scatter(pn1_v, [slot_s], e1, mask=mask)
                    plsc.store_compressed(srcR.at[pl.ds(off, L)], src16,
                                          mask=mask)
                    plsc.store_compressed(slotR.at[pl.ds(off, L)], slot_s,
                                          mask=mask)
                    plsc.store_compressed(e0R.at[pl.ds(off, L)], e0, mask=mask)
                    plsc.store_compressed(e1R.at[pl.ds(off, L)], e1, mask=mask)
                return off + cnt
            return lax.fori_loop(0, CH // L, gbody, off)

        off = lax.fori_loop(0, EC // CH, chunk_body, jnp.int32(0))

        srcR[pl.ds(off, L)] = jnp.zeros((L,), jnp.int32)
        slotR[pl.ds(off, L)] = jnp.full((L,), DUMP, jnp.int32)
        e0R[pl.ds(off, L)] = zf
        e1R[pl.ds(off, L)] = zf

        ng = (off + L - 1) // L

        def _start(j, buf, gsem):
            src16 = srcR[pl.ds(j * L, L)]
            pltpu.async_copy(x_hbm.at[src16], buf, gsem)

        def _drain(j, buf, gsem):
            src16 = srcR[pl.ds(j * L, L)]
            pltpu.make_async_copy(x_hbm.at[src16], buf, gsem).wait()

        def _consume(j, buf):
            o = j * L
            slot16 = slotR[pl.ds(o, L)]
            for k in range(L):
                kk = jnp.full((L,), o + k, jnp.int32)
                e0b = plsc.load_gather(e0R, [kk])
                e1b = plsc.load_gather(e1R, [kk])
                for dd in range(D // L):
                    v = buf[k, pl.ds(dd * L, L)]
                    s0_v[k, pl.ds(dd * L, L)] = v * e0b
                    s1_v[k, pl.ds(dd * L, L)] = v * e1b
            pltpu.async_copy(s0_v, agg0_sp.at[slot16], sem, add=True)
            pltpu.async_copy(s1_v, agg1_sp.at[slot16], sem, add=True)
            pltpu.make_async_copy(s0_v, agg0_sp.at[slot16], sem).wait()
            pltpu.make_async_copy(s1_v, agg1_sp.at[slot16], sem).wait()

        @pl.when(ng > 0)
        def _():
            _start(0, rows0_v, gsem0)

        def rbody(jj, c):
            j0 = jj * 2
            j1 = j0 + 1

            @pl.when(j1 < ng)
            def _():
                _start(j1, rows1_v, gsem1)

            @pl.when(j0 < ng)
            def _():
                _drain(j0, rows0_v, gsem0)
                _consume(j0, rows0_v)

            @pl.when(j0 + 2 < ng)
            def _():
                _start(j0 + 2, rows0_v, gsem0)

            @pl.when(j1 < ng)
            def _():
                _drain(j1, rows1_v, gsem1)
                _consume(j1, rows1_v)
            return c
        lax.fori_loop(0, (ng + 1) // 2, rbody, 0)

        plsc.subcore_barrier()
        pltpu.sync_copy(agg0_sp.at[pl.ds(sid * rs, rs)],
                        agg_out.at[cid, 0, pl.ds(sid * rs, rs)])
        pltpu.sync_copy(agg1_sp.at[pl.ds(sid * rs, rs)],
                        agg_out.at[cid, 1, pl.ds(sid * rs, rs)])
        pltpu.sync_copy(pn0_v, pn_out.at[wid, 0])
        pltpu.sync_copy(pn1_v, pn_out.at[wid, 1])

    return kern


def _build_kernel_d(interpret=False):
    def body(xb_ref, soe_col_ref, aggp_ref, pnp_ref, wr_ref,
             wu_ref, wt_ref, bias_ref, pall_ref, p_ref):
        f32 = jnp.float32
        xb = xb_ref[...]
        soe_col = soe_col_ref[...]
        aggp = aggp_ref[...]
        pn = jnp.sum(pnp_ref[...], axis=0)
        wr = wr_ref[...]
        wu = wu_ref[...]
        wt = wt_ref[...]
        bias = bias_ref[...]

        slot_iota = lax.broadcasted_iota(jnp.int32, (B, SLOTS), 1)
        onehot = (soe_col == slot_iota).astype(f32)
        wt8 = jnp.concatenate([wt, jnp.zeros((8 - G, D), f32)], axis=0)
        thr_all = lax.dot_general(xb, wt8, (((1,), (1,)), ((), ())),
                                  preferred_element_type=f32)

        p_cols = []
        pall_cols = []
        for g in range(G):
            agg_g = aggp[0, g] + aggp[1, g]
            agg_e = lax.dot_general(onehot, agg_g,
                                    (((1,), (0,)), ((), ())),
                                    preferred_element_type=f32,
                                    precision=lax.Precision.HIGHEST)
            a = agg_e * wu[g:g + 1]
            h = xb * wr[g:g + 1] + bias[g:g + 1]
            num = jnp.sum(a * h, axis=1, keepdims=True)
            na = jnp.sqrt(jnp.sum(a * a, axis=1, keepdims=True))
            nh = jnp.sqrt(jnp.sum(h * h, axis=1, keepdims=True))
            cos = num / (na * nh + 1e-6)
            pn_e = lax.dot_general(onehot, pn[g:g + 1],
                                   (((1,), (1,)), ((), ())),
                                   preferred_element_type=f32,
                                   precision=lax.Precision.HIGHEST)
            pcol = 0.5 * cos + 0.5 * jax.nn.sigmoid(pn_e)
            thr = jax.nn.sigmoid(thr_all[:, g:g + 1])
            pall_g = jnp.where(pcol > thr, pcol, 0.0)
            pall_cols.append(pall_g)

        pall_ref[...] = jnp.concatenate(pall_cols, axis=1)
        p_ref[...] = pall_cols[0] + pall_cols[1]

    return pl.pallas_call(
        body,
        out_shape=[
            jax.ShapeDtypeStruct((B, G), jnp.float32),
            jax.ShapeDtypeStruct((B, 1), jnp.float32),
        ],
        interpret=interpret,
    )


def _build_kernel_e(interpret=False):
    def body(pc_ref, pr_ref, sc_ref, sr_ref, bp_ref, bi_ref):
        f32 = jnp.float32
        p_col = pc_ref[...]
        p_row = pr_ref[...]
        soe_col = sc_ref[...]
        soe_row = sr_ref[...]
        ii = lax.broadcasted_iota(jnp.int32, (B, B), 0)
        jj = lax.broadcasted_iota(jnp.int32, (B, B), 1)
        same_slot = soe_col == soe_row
        idx_lt = ii < jj
        beats = (jnp.logical_not(same_slot)
                 & ((p_col > p_row) | ((p_col == p_row) & idx_lt))
                 | (same_slot & idx_lt))
        rank_row = jnp.sum(beats.astype(f32), axis=0, keepdims=True)
        r_iota = lax.broadcasted_iota(jnp.int32, (KB, B), 0).astype(f32)
        sel = (r_iota == rank_row).astype(f32)
        bp_ref[...] = jnp.sum(sel * p_row, axis=1, keepdims=True)
        idx_row = lax.broadcasted_iota(jnp.int32, (1, B), 1).astype(f32)
        bi_ref[...] = jnp.sum(sel * idx_row, axis=1,
                              keepdims=True).astype(jnp.int32)

    return pl.pallas_call(
        body,
        out_shape=[
            jax.ShapeDtypeStruct((KB, 1), jnp.float32),
            jax.ShapeDtypeStruct((KB, 1), jnp.int32),
        ],
        interpret=interpret,
    )


_kernel_a = _build_kernel_a()
_kernel_b = _build_kernel_b()
_kernel_c = _build_kernel_c()
_kernel_d = _build_kernel_d()
_kernel_e = _build_kernel_e()


def kernel(x, w_ego_root, w_ego_u, w_layer_v, w_layer_u, w_threshold, bias,
           edge_index, batch_nodes):
    src = edge_index[0]
    dst = edge_index[1]
    deg_parts, soe, xb = _kernel_a(dst, batch_nodes, x)
    w8 = jnp.concatenate(
        [w_layer_u[0:1], w_layer_v[0:1], w_layer_u[1:2], w_layer_v[1:2],
         jnp.zeros((4, D), jnp.float32)], axis=0)
    lv8, nimp = _kernel_b(x, w8, deg_parts)
    zrows = jnp.zeros((SLOTS // 16, D), jnp.float32)
    agg_parts, pn_parts = _kernel_c(src, dst, lv8, nimp, batch_nodes, x,
                                    zrows)
    pall_col, p_col = _kernel_d(xb, soe.reshape(B, 1), agg_parts, pn_parts,
                                w_ego_root, w_ego_u, w_threshold, bias)
    bp, bi = _kernel_e(p_col, p_col.reshape(1, B),
                       soe.reshape(B, 1), soe.reshape(1, B))
    return (jnp.transpose(pall_col), bp.reshape(KB), bi.reshape(KB))


# speedup vs baseline: 250.5775x; 1.0036x over previous
import functools

import jax
import jax.numpy as jnp
from jax import lax
from jax.experimental import pallas as pl
from jax.experimental.pallas import tpu as pltpu
from jax.experimental.pallas import tpu_sc as plsc

N = 10000
E = 320000
D = 128
G = 2
B = 512
KB = 200
NP_ = 10240
SLOTS = 640
DUMP = 512
NTILES = 32
EC = E // NTILES
CH = 2000
CAP = EC + 16
L = 16

_sc_mesh = plsc.VectorSubcoreMesh(core_axis_name="c", subcore_axis_name="s",
                                  num_cores=2, num_subcores=16)


def _build_kernel_a(interpret=False):
    @functools.partial(
        pl.kernel,
        out_type=[
            jax.ShapeDtypeStruct((NTILES, NP_), jnp.float32),
            jax.ShapeDtypeStruct((B,), jnp.int32),
            jax.ShapeDtypeStruct((B, D), jnp.float32),
        ],
        mesh=_sc_mesh,
        compiler_params=pltpu.CompilerParams(needs_layout_passes=False, use_tc_tiling_on_sc=False),
        scratch_types=[
            pltpu.VMEM((NP_,), jnp.float32),
            pltpu.VMEM((EC,), jnp.int32),
            pltpu.VMEM((N,), jnp.int32),
            pltpu.VMEM((B,), jnp.int32),
            pltpu.VMEM((B,), jnp.int32),
            pltpu.VMEM((L,), jnp.int32),
            pltpu.VMEM((L, D), jnp.float32),
            pltpu.SemaphoreType.DMA,
        ],
        interpret=interpret,
    )
    def kern(dst_hbm, batch_hbm, x_hbm, deg_out, soe_out, xb_out,
             deg_v, dst_v, n2s_v, batch_v, soe_v, idx_v, rows_v, sem):
        cid = lax.axis_index("c")
        sid = lax.axis_index("s")
        wid = cid * 16 + sid
        zf = jnp.zeros((L,), jnp.float32)

        def zbody(i, c):
            deg_v[pl.ds(i * L, L)] = zf
            return c
        lax.fori_loop(0, NP_ // L, zbody, 0)

        pltpu.sync_copy(dst_hbm.at[pl.ds(wid * EC, EC)], dst_v)
        onesf = jnp.ones((L,), jnp.float32)

        def sbody(i, c):
            idx = dst_v[pl.ds(i * L, L)]
            plsc.addupdate_scatter(deg_v, [idx], onesf)
            return c
        lax.fori_loop(0, EC // L, sbody, 0)
        pltpu.sync_copy(deg_v, deg_out.at[wid])

        pltpu.sync_copy(batch_hbm.at[pl.ds(wid * L, L)], idx_v)
        pltpu.async_copy(x_hbm.at[idx_v], rows_v, sem).wait()
        pltpu.sync_copy(rows_v, xb_out.at[pl.ds(wid * L, L)])

        @pl.when(wid == 0)
        def _():
            pltpu.sync_copy(batch_hbm, batch_v)
            neg1 = jnp.full((L,), -1, jnp.int32)

            def mbody(i, c):
                n2s_v[pl.ds(i * L, L)] = neg1
                return c
            lax.fori_loop(0, N // L, mbody, 0)
            iota = lax.iota(jnp.int32, L)

            def scb(i, c):
                bidx = batch_v[pl.ds(i * L, L)]
                plsc.store_scatter(n2s_v, [bidx], iota + i * L)
                return c
            lax.fori_loop(0, B // L, scb, 0)

            def gab(i, c):
                bidx = batch_v[pl.ds(i * L, L)]
                soe_v[pl.ds(i * L, L)] = plsc.load_gather(n2s_v, [bidx])
                return c
            lax.fori_loop(0, B // L, gab, 0)
            pltpu.sync_copy(soe_v, soe_out)

    return kern


def _build_kernel_b(interpret=False):
    def body(x_ref, w8_ref, deg_ref, lv_ref, nimp_ref):
        xb = x_ref[...]
        w8 = w8_ref[...]
        lv_ref[...] = lax.dot_general(
            w8, xb, (((1,), (1,)), ((), ())),
            preferred_element_type=jnp.float32)
        deg = jnp.sum(deg_ref[...], axis=0, keepdims=True)
        nimp_ref[...] = 1.0 / jnp.maximum(deg, 1.0)

    return pl.pallas_call(
        body,
        grid=(10,),
        in_specs=[
            pl.BlockSpec((NP_ // 10, D), lambda i: (i, 0)),
            pl.BlockSpec((8, D), lambda i: (0, 0)),
            pl.BlockSpec((NTILES, NP_ // 10), lambda i: (0, i)),
        ],
        out_specs=[
            pl.BlockSpec((8, NP_ // 10), lambda i: (0, i)),
            pl.BlockSpec((1, NP_ // 10), lambda i: (0, i)),
        ],
        out_shape=[
            jax.ShapeDtypeStruct((8, NP_), jnp.float32),
            jax.ShapeDtypeStruct((1, NP_), jnp.float32),
        ],
        interpret=interpret,
    )


def _build_kernel_c(interpret=False):
    @functools.partial(
        pl.kernel,
        out_type=[
            jax.ShapeDtypeStruct((2, G, SLOTS, D), jnp.float32),
            jax.ShapeDtypeStruct((NTILES, G, SLOTS), jnp.float32),
        ],
        mesh=_sc_mesh,
        compiler_params=pltpu.CompilerParams(needs_layout_passes=False, use_tc_tiling_on_sc=False),
        scratch_types=[
            pltpu.VMEM((N,), jnp.float32),
            pltpu.VMEM((N,), jnp.float32),
            pltpu.VMEM((N,), jnp.float32),
            pltpu.VMEM((N,), jnp.float32),
            pltpu.VMEM((N,), jnp.float32),
            pltpu.VMEM((N,), jnp.int32),
            pltpu.VMEM((B,), jnp.int32),
            pltpu.VMEM((CH,), jnp.int32),
            pltpu.VMEM((CH,), jnp.int32),
            pltpu.VMEM((CAP,), jnp.int32),
            pltpu.VMEM((CAP,), jnp.int32),
            pltpu.VMEM((CAP,), jnp.float32),
            pltpu.VMEM((CAP,), jnp.float32),
            pltpu.VMEM((L, D), jnp.float32),
            pltpu.VMEM((L, D), jnp.float32),
            pltpu.VMEM((L, D), jnp.float32),
            pltpu.VMEM((L, D), jnp.float32),
            pltpu.VMEM((SLOTS,), jnp.float32),
            pltpu.VMEM((SLOTS,), jnp.float32),
            pltpu.SemaphoreType.DMA,
            pltpu.SemaphoreType.DMA,
            pltpu.SemaphoreType.DMA,
            pltpu.VMEM_SHARED((SLOTS, D), jnp.float32),
            pltpu.VMEM_SHARED((SLOTS, D), jnp.float32),
        ],
        interpret=interpret,
    )
    def kern(src_hbm, dst_hbm, lv_hbm, nimp_hbm, batch_hbm, x_hbm, zrow_hbm,
             agg_out, pn_out,
             lu0_v, lv0_v, lu1_v, lv1_v, nimp_v, n2s_v, batch_v,
             srcc_v, dstc_v, srcR, slotR, e0R, e1R,
             rows0_v, rows1_v, s0_v, s1_v, pn0_v, pn1_v,
             sem, gsem0, gsem1, agg0_sp, agg1_sp):
        cid = lax.axis_index("c")
        sid = lax.axis_index("s")
        wid = cid * 16 + sid
        zf = jnp.zeros((L,), jnp.float32)

        pltpu.async_copy(lv_hbm.at[0, pl.ds(0, N)], lu0_v, sem)
        pltpu.async_copy(lv_hbm.at[1, pl.ds(0, N)], lv0_v, sem)
        pltpu.async_copy(lv_hbm.at[2, pl.ds(0, N)], lu1_v, sem)
        pltpu.async_copy(lv_hbm.at[3, pl.ds(0, N)], lv1_v, sem)
        pltpu.async_copy(nimp_hbm.at[0, pl.ds(0, N)], nimp_v, sem)
        pltpu.async_copy(batch_hbm, batch_v, sem)
        pltpu.make_async_copy(lv_hbm.at[0, pl.ds(0, N)], lu0_v, sem).wait()
        pltpu.make_async_copy(lv_hbm.at[1, pl.ds(0, N)], lv0_v, sem).wait()
        pltpu.make_async_copy(lv_hbm.at[2, pl.ds(0, N)], lu1_v, sem).wait()
        pltpu.make_async_copy(lv_hbm.at[3, pl.ds(0, N)], lv1_v, sem).wait()
        pltpu.make_async_copy(nimp_hbm.at[0, pl.ds(0, N)], nimp_v, sem).wait()
        pltpu.make_async_copy(batch_hbm, batch_v, sem).wait()

        neg1 = jnp.full((L,), -1, jnp.int32)

        def mbody(i, c):
            n2s_v[pl.ds(i * L, L)] = neg1
            return c
        lax.fori_loop(0, N // L, mbody, 0)
        iota = lax.iota(jnp.int32, L)

        def scb(i, c):
            bidx = batch_v[pl.ds(i * L, L)]
            plsc.store_scatter(n2s_v, [bidx], iota + i * L)
            return c
        lax.fori_loop(0, B // L, scb, 0)

        def pzb(i, c):
            pn0_v[pl.ds(i * L, L)] = zf
            pn1_v[pl.ds(i * L, L)] = zf
            return c
        lax.fori_loop(0, SLOTS // L, pzb, 0)
        rs = SLOTS // 16
        pltpu.sync_copy(zrow_hbm, agg0_sp.at[pl.ds(sid * rs, rs)])
        pltpu.sync_copy(zrow_hbm, agg1_sp.at[pl.ds(sid * rs, rs)])
        plsc.subcore_barrier()

        def chunk_body(cix, off):
            base = wid * EC + cix * CH
            pltpu.async_copy(src_hbm.at[pl.ds(base, CH)], srcc_v, sem)
            pltpu.async_copy(dst_hbm.at[pl.ds(base, CH)], dstc_v, sem)
            pltpu.make_async_copy(src_hbm.at[pl.ds(base, CH)], srcc_v,
                                  sem).wait()
            pltpu.make_async_copy(dst_hbm.at[pl.ds(base, CH)], dstc_v,
                                  sem).wait()

            def gbody(i, off):
                dst16 = dstc_v[pl.ds(i * L, L)]
                slot16 = plsc.load_gather(n2s_v, [dst16])
                mask = slot16 >= 0
                cnt = jnp.sum(jnp.where(mask, 1, 0))

                @pl.when(cnt > 0)
                def _():
                    src16 = srcc_v[pl.ds(i * L, L)]
                    ni = plsc.load_gather(nimp_v, [src16])
                    a0 = (plsc.load_gather(lu0_v, [src16])
                          + plsc.load_gather(lv0_v, [dst16]))
                    a1 = (plsc.load_gather(lu1_v, [src16])
                          + plsc.load_gather(lv1_v, [dst16]))
                    e0 = (1.0 / (1.0 + jnp.exp(-a0))) * ni
                    e1 = (1.0 / (1.0 + jnp.exp(-a1))) * ni
                    slot_s = jnp.where(mask, slot16, DUMP)
                    plsc.addupdate_scatter(pn0_v, [slot_s], e0, mask=mask)
                    plsc.addupdate_scatter(pn1_v, [slot_s], e1, mask=mask)
                    plsc.store_compressed(srcR.at[pl.ds(off, L)], src16,
                                          mask=mask)
                    plsc.store_compressed(slotR.at[pl.ds(off, L)], slot_s,
                                          mask=mask)
                    plsc.store_compressed(e0R.at[pl.ds(off, L)], e0, mask=mask)
                    plsc.store_compressed(e1R.at[pl.ds(off, L)], e1, mask=mask)
                return off + cnt
            return lax.fori_loop(0, CH // L, gbody, off)

        off = lax.fori_loop(0, EC // CH, chunk_body, jnp.int32(0))

        srcR[pl.ds(off, L)] = jnp.zeros((L,), jnp.int32)
        slotR[pl.ds(off, L)] = jnp.full((L,), DUMP, jnp.int32)
        e0R[pl.ds(off, L)] = zf
        e1R[pl.ds(off, L)] = zf

        ng = (off + L - 1) // L

        def _start(j, buf, gsem):
            src16 = srcR[pl.ds(j * L, L)]
            pltpu.async_copy(x_hbm.at[src16], buf, gsem)

        def _drain(j, buf, gsem):
            src16 = srcR[pl.ds(j * L, L)]
            pltpu.make_async_copy(x_hbm.at[src16], buf, gsem).wait()

        def _consume(j, buf):
            o = j * L
            slot16 = slotR[pl.ds(o, L)]
            for k in range(L):
                kk = jnp.full((L,), o + k, jnp.int32)
                e0b = plsc.load_gather(e0R, [kk])
                e1b = plsc.load_gather(e1R, [kk])
                for dd in range(D // L):
                    v = buf[k, pl.ds(dd * L, L)]
                    s0_v[k, pl.ds(dd * L, L)] = v * e0b
                    s1_v[k, pl.ds(dd * L, L)] = v * e1b
            pltpu.async_copy(s0_v, agg0_sp.at[slot16], sem, add=True)
            pltpu.async_copy(s1_v, agg1_sp.at[slot16], sem, add=True)
            pltpu.make_async_copy(s0_v, agg0_sp.at[slot16], sem).wait()
            pltpu.make_async_copy(s1_v, agg1_sp.at[slot16], sem).wait()

        @pl.when(ng > 0)
        def _():
            _start(0, rows0_v, gsem0)

        def rbody(jj, c):
            j0 = jj * 2
            j1 = j0 + 1

            @pl.when(j1 < ng)
            def _():
                _start(j1, rows1_v, gsem1)

            @pl.when(j0 < ng)
            def _():
                _drain(j0, rows0_v, gsem0)
                _consume(j0, rows0_v)

            @pl.when(j0 + 2 < ng)
            def _():
                _start(j0 + 2, rows0_v, gsem0)

            @pl.when(j1 < ng)
            def _():
                _drain(j1, rows1_v, gsem1)
                _consume(j1, rows1_v)
            return c
        lax.fori_loop(0, (ng + 1) // 2, rbody, 0)

        plsc.subcore_barrier()
        pltpu.sync_copy(agg0_sp.at[pl.ds(sid * rs, rs)],
                        agg_out.at[cid, 0, pl.ds(sid * rs, rs)])
        pltpu.sync_copy(agg1_sp.at[pl.ds(sid * rs, rs)],
                        agg_out.at[cid, 1, pl.ds(sid * rs, rs)])
        pltpu.sync_copy(pn0_v, pn_out.at[wid, 0])
        pltpu.sync_copy(pn1_v, pn_out.at[wid, 1])

    return kern


def _build_kernel_d(interpret=False):
    def body(xb_ref, soe_col_ref, soe_row_ref, aggp_ref, pnp_ref, wr_ref,
             wu_ref, wt_ref, bias_ref, pall_ref, bp_ref, bi_ref):
        f32 = jnp.float32
        xb = xb_ref[...]
        soe_col = soe_col_ref[...]
        aggp = aggp_ref[...]
        pn = jnp.sum(pnp_ref[...], axis=0)
        wr = wr_ref[...]
        wu = wu_ref[...]
        wt = wt_ref[...]
        bias = bias_ref[...]

        slot_iota = lax.broadcasted_iota(jnp.int32, (B, SLOTS), 1)
        onehot = (soe_col == slot_iota).astype(f32)
        wt8 = jnp.concatenate([wt, jnp.zeros((8 - G, D), f32)], axis=0)
        thr_all = lax.dot_general(xb, wt8, (((1,), (1,)), ((), ())),
                                  preferred_element_type=f32)

        p_cols = []
        pall_cols = []
        for g in range(G):
            agg_g = aggp[0, g] + aggp[1, g]
            agg_e = lax.dot_general(onehot, agg_g,
                                    (((1,), (0,)), ((), ())),
                                    preferred_element_type=f32,
                                    precision=lax.Precision.HIGHEST)
            a = agg_e * wu[g:g + 1]
            h = xb * wr[g:g + 1] + bias[g:g + 1]
            num = jnp.sum(a * h, axis=1, keepdims=True)
            na = jnp.sqrt(jnp.sum(a * a, axis=1, keepdims=True))
            nh = jnp.sqrt(jnp.sum(h * h, axis=1, keepdims=True))
            cos = num / (na * nh + 1e-6)
            pn_e = lax.dot_general(onehot, pn[g:g + 1],
                                   (((1,), (1,)), ((), ())),
                                   preferred_element_type=f32,
                                   precision=lax.Precision.HIGHEST)
            pcol = 0.5 * cos + 0.5 * jax.nn.sigmoid(pn_e)
            thr = jax.nn.sigmoid(thr_all[:, g:g + 1])
            pall_g = jnp.where(pcol > thr, pcol, 0.0)
            pall_cols.append(pall_g)

        pall_ref[...] = jnp.concatenate(pall_cols, axis=1)
        p_col = pall_cols[0] + pall_cols[1]
        p_row = jnp.reshape(p_col, (1, B))
        soe_row = soe_row_ref[...]
        ii = lax.broadcasted_iota(jnp.int32, (B, B), 0)
        jj = lax.broadcasted_iota(jnp.int32, (B, B), 1)
        same_slot = soe_col == soe_row
        idx_lt = ii < jj
        beats = (jnp.logical_not(same_slot)
                 & ((p_col > p_row) | ((p_col == p_row) & idx_lt))
                 | (same_slot & idx_lt))
        rank_row = jnp.sum(beats.astype(f32), axis=0, keepdims=True)
        r_iota = lax.broadcasted_iota(jnp.int32, (KB, B), 0).astype(f32)
        sel = (r_iota == rank_row).astype(f32)
        bp_ref[...] = jnp.sum(sel * p_row, axis=1, keepdims=True)
        idx_row = lax.broadcasted_iota(jnp.int32, (1, B), 1).astype(f32)
        bi_ref[...] = jnp.sum(sel * idx_row, axis=1,
                              keepdims=True).astype(jnp.int32)

    return pl.pallas_call(
        body,
        out_shape=[
            jax.ShapeDtypeStruct((B, G), jnp.float32),
            jax.ShapeDtypeStruct((KB, 1), jnp.float32),
            jax.ShapeDtypeStruct((KB, 1), jnp.int32),
        ],
        interpret=interpret,
    )


_kernel_a = _build_kernel_a()
_kernel_b = _build_kernel_b()
_kernel_c = _build_kernel_c()
_kernel_d = _build_kernel_d()


def kernel(x, w_ego_root, w_ego_u, w_layer_v, w_layer_u, w_threshold, bias,
           edge_index, batch_nodes):
    src = edge_index[0]
    dst = edge_index[1]
    deg_parts, soe, xb = _kernel_a(dst, batch_nodes, x)
    w8 = jnp.concatenate(
        [w_layer_u[0:1], w_layer_v[0:1], w_layer_u[1:2], w_layer_v[1:2],
         jnp.zeros((4, D), jnp.float32)], axis=0)
    lv8, nimp = _kernel_b(x, w8, deg_parts)
    zrows = jnp.zeros((SLOTS // 16, D), jnp.float32)
    agg_parts, pn_parts = _kernel_c(src, dst, lv8, nimp, batch_nodes, x,
                                    zrows)
    pall_col, bp, bi = _kernel_d(xb, soe.reshape(B, 1), soe.reshape(1, B),
                                 agg_parts, pn_parts,
                                 w_ego_root, w_ego_u, w_threshold, bias)
    return (jnp.transpose(pall_col), bp.reshape(KB), bi.reshape(KB))
